# Initial kernel scaffold; baseline (speedup 1.0000x reference)
import jax
import jax.numpy as jnp
from jax.experimental import pallas as pl


def kernel(query, value, rand_matrix, seed):
    raise NotImplementedError("write your pallas kernel here")


# speedup vs baseline: 27.3869x; 27.3869x over previous
import functools
import math

import jax
import jax.numpy as jnp
from jax.experimental import pallas as pl

D_KK = 64
N_ROUNDS = 4
BKT = 64
SEQ = 4096
NBKT = SEQ // BKT
NBB = 8
NBLK = NBKT // NBB


def _attn_body(rq, rqh, rv, rvh, qik, qikh, qic, sqhk, sqhkh, sqhc,
               sbqk, sbqkh, sbqc, att, lse):
    q = rq[0, 0]
    qh = rqh[0, 0]
    v = rv[0, 0]
    vh = rvh[0, 0]
    qi_rows = qik[0, 0, :, 0]
    qi_prev0 = qikh[0, 0, 0]
    qi_col = qic[0, 0]
    sq_rows = sqhk[0, 0, :, 0]
    sq_prev0 = sqhkh[0, 0, 0]
    sq_col = sqhc[0, 0]
    sb_rows = sbqk[0, 0, :, 0]
    sb_prev0 = sbqkh[0, 0, 0]
    sb_col = sbqc[0, 0]

    inv_sqrt_d = 1.0 / math.sqrt(D_KK)
    lse_cols = []
    for t in range(NBB):
        qt = q[t * BKT:(t + 1) * BKT, :]
        prev = qh if t == 0 else q[(t - 1) * BKT:t * BKT, :]
        kw = jnp.concatenate([prev, qt], axis=0)
        nrm = jnp.sqrt(jnp.sum(kw * kw, axis=1, keepdims=True))
        kn = kw / jnp.maximum(nrm, 1e-12)
        s = jax.lax.dot_general(
            qt, kn, (((1,), (1,)), ((), ())),
            preferred_element_type=jnp.float32) * inv_sqrt_d

        qi_q = qi_col[t * BKT:(t + 1) * BKT, :]
        qi_p = qi_prev0 if t == 0 else qi_rows[t - 1:t, :]
        qi_k = jnp.concatenate([qi_p, qi_rows[t:t + 1, :]], axis=1)
        sq_q = sq_col[t * BKT:(t + 1) * BKT, :]
        sq_p = sq_prev0 if t == 0 else sq_rows[t - 1:t, :]
        sq_k = jnp.concatenate([sq_p, sq_rows[t:t + 1, :]], axis=1)
        sb_q = sb_col[t * BKT:(t + 1) * BKT, :]
        sb_p = sb_prev0 if t == 0 else sb_rows[t - 1:t, :]
        sb_k = jnp.concatenate([sb_p, sb_rows[t:t + 1, :]], axis=1)

        s = jnp.where(sq_q != sq_k, -1e9, s)
        s = jnp.where(qi_q < qi_k, -1e9, s)
        s = jnp.where(qi_q == qi_k, -1e5, s)

        cnt = jnp.zeros(s.shape, jnp.int32)
        for rp in range(N_ROUNDS):
            aa = (sb_q >> (8 * rp)) & 255
            bb = (sb_k >> (8 * rp)) & 255
            cnt = cnt + jnp.where(((aa - bb) & (NBKT - 1)) <= 1, 1, 0)

        m = jnp.max(s, axis=1, keepdims=True)
        e = jnp.exp(s - m)
        ssum = jnp.sum(e, axis=1, keepdims=True)
        lse_cols.append(m + jnp.log(ssum))
        p = e / (ssum * cnt.astype(jnp.float32))

        vw = jnp.concatenate([vh if t == 0 else v[(t - 1) * BKT:t * BKT, :],
                              v[t * BKT:(t + 1) * BKT, :]], axis=0)
        ot = jax.lax.dot_general(
            p, vw, (((1,), (0,)), ((), ())),
            preferred_element_type=jnp.float32)
        att[0, 0, t * BKT:(t + 1) * BKT, :] = ot
    lse[0, 0, 0] = jnp.concatenate(lse_cols, axis=1)


def _combine_body(lse_ref, att_ref, out_ref):
    x = lse_ref[0]
    m = jnp.max(x, axis=0, keepdims=True)
    e = jnp.exp(x - m)
    w = e / jnp.sum(e, axis=0, keepdims=True)
    a = att_ref[0]
    acc = a[0] * w[:, 0:1]
    for r in range(1, N_ROUNDS):
        acc = acc + a[r] * w[:, r:r + 1]
    out_ref[0] = acc


@jax.jit
def kernel(query, value, rand_matrix, seed):
    B, L, D = query.shape
    R = N_ROUNDS

    qn = query / jnp.maximum(
        jnp.sqrt(jnp.sum(query * query, axis=-1, keepdims=True)), 1e-12)
    mm = jnp.einsum('bld,bdrk->blrk', qn, rand_matrix)
    h = jnp.argmax(jnp.concatenate([mm, -mm], axis=-1), axis=-1).astype(jnp.int32)

    keys = h * L + jnp.arange(L, dtype=jnp.int32)[None, :, None]
    hash_indice = jnp.argsort(keys, axis=1).astype(jnp.int32)
    original_indice = jnp.argsort(hash_indice, axis=1).astype(jnp.int32)

    sb = original_indice // BKT
    sbpack = jnp.zeros((B, L), jnp.int32)
    for rp in range(R):
        sbpack = sbpack | (sb[:, :, rp] << (8 * rp))

    hi_t = jnp.transpose(hash_indice, (0, 2, 1))
    oi_t = jnp.transpose(original_indice, (0, 2, 1))

    rqg = jnp.take_along_axis(
        jnp.broadcast_to(query[:, None], (B, R, L, D)), hi_t[..., None], axis=2)
    rvg = jnp.take_along_axis(
        jnp.broadcast_to(value[:, None], (B, R, L, D)), hi_t[..., None], axis=2)
    h_t = jnp.transpose(h, (0, 2, 1))
    sqh = jnp.take_along_axis(h_t, hi_t, axis=2)
    sbq = jnp.take_along_axis(
        jnp.broadcast_to(sbpack[:, None], (B, R, L)), hi_t, axis=2)

    qik = hi_t.reshape(B, R, NBKT, 1, BKT)
    sqhk = sqh.reshape(B, R, NBKT, 1, BKT)
    sbqk = sbq.reshape(B, R, NBKT, 1, BKT)
    qic = hi_t.reshape(B, R, L, 1)
    sqhc = sqh.reshape(B, R, L, 1)
    sbqc = sbq.reshape(B, R, L, 1)

    grid = (B, R, NBLK)

    def blk(b, r, n):
        return (b, r, n, 0)

    def blk_halo(b, r, n):
        return (b, r, (n * NBB - 1) % NBKT, 0)

    def blk5(b, r, n):
        return (b, r, n, 0, 0)

    def blk5_halo(b, r, n):
        return (b, r, (n * NBB - 1) % NBKT, 0, 0)

    att, lse5 = pl.pallas_call(
        _attn_body,
        grid=grid,
        in_specs=[
            pl.BlockSpec((1, 1, NBB * BKT, D), blk),
            pl.BlockSpec((1, 1, BKT, D), blk_halo),
            pl.BlockSpec((1, 1, NBB * BKT, D), blk),
            pl.BlockSpec((1, 1, BKT, D), blk_halo),
            pl.BlockSpec((1, 1, NBB, 1, BKT), blk5),
            pl.BlockSpec((1, 1, 1, 1, BKT), blk5_halo),
            pl.BlockSpec((1, 1, NBB * BKT, 1), blk),
            pl.BlockSpec((1, 1, NBB, 1, BKT), blk5),
            pl.BlockSpec((1, 1, 1, 1, BKT), blk5_halo),
            pl.BlockSpec((1, 1, NBB * BKT, 1), blk),
            pl.BlockSpec((1, 1, NBB, 1, BKT), blk5),
            pl.BlockSpec((1, 1, 1, 1, BKT), blk5_halo),
            pl.BlockSpec((1, 1, NBB * BKT, 1), blk),
        ],
        out_specs=[
            pl.BlockSpec((1, 1, NBB * BKT, D), blk),
            pl.BlockSpec((1, 1, 1, BKT, NBB), lambda b, r, n: (b, r, n, 0, 0)),
        ],
        out_shape=[
            jax.ShapeDtypeStruct((B, R, L, D), jnp.float32),
            jax.ShapeDtypeStruct((B, R, NBLK, BKT, NBB), jnp.float32),
        ],
    )(rqg, rqg, rvg, rvg, qik, qik, qic, sqhk, sqhk, sqhc,
      sbqk, sbqk, sbqc)

    lse_sorted = jnp.transpose(lse5, (0, 1, 2, 4, 3)).reshape(B, R, L)

    att_orig = jnp.take_along_axis(att, oi_t[..., None], axis=2)
    lse_orig = jnp.take_along_axis(lse_sorted, oi_t, axis=2)
    lse_blr = jnp.transpose(lse_orig, (0, 2, 1))

    out = pl.pallas_call(
        _combine_body,
        grid=(B,),
        in_specs=[
            pl.BlockSpec((1, L, R), lambda b: (b, 0, 0)),
            pl.BlockSpec((1, R, L, D), lambda b: (b, 0, 0, 0)),
        ],
        out_specs=pl.BlockSpec((1, L, D), lambda b: (b, 0, 0)),
        out_shape=jax.ShapeDtypeStruct((B, L, D), jnp.float32),
    )(lse_blr, att_orig)
    return out
